# Initial kernel scaffold; baseline (speedup 1.0000x reference)
import jax
import jax.numpy as jnp
from jax.experimental import pallas as pl


def kernel(route_id, node_id, weekday, dense_feats, route_table, node_table, weekday_table, W1, b1, W2, b2, W3, b3):
    raise NotImplementedError("write your pallas kernel here")


# speedup vs baseline: 1.1727x; 1.1727x over previous
import functools

import jax
import jax.numpy as jnp
from jax import lax
from jax.experimental import pallas as pl
from jax.experimental.pallas import tpu as pltpu
from jax.experimental.pallas import tpu_sc as plsc

_B = 16384
_NC = 2
_NS = 16
_NW = _NC * _NS
_CHUNK = 128
_IDX_ROWS = (3 * _B) // _CHUNK
_RPW = _IDX_ROWS // _NW
_D = 16


def _sc_gather(table, idx):
    mesh = plsc.VectorSubcoreMesh(core_axis_name="c", subcore_axis_name="s")
    bpw = _RPW * _CHUNK

    @functools.partial(
        pl.kernel,
        out_type=jax.ShapeDtypeStruct((_IDX_ROWS * _CHUNK, _D), jnp.float32),
        mesh=mesh,
        scratch_types=[
            pltpu.VMEM((bpw,), jnp.int32),
            pltpu.VMEM((bpw, _D), jnp.float32),
            pltpu.SemaphoreType.DMA,
        ],
        compiler_params=pltpu.CompilerParams(use_tc_tiling_on_sc=False),
    )
    def gather_kernel(table_hbm, idx_hbm, out_hbm, idx_v, rows_v, sem):
        wid = lax.axis_index("s") * _NC + lax.axis_index("c")
        base = wid * bpw
        pltpu.sync_copy(idx_hbm.at[pl.ds(base, bpw)], idx_v)
        copies = []
        for j in range(_RPW):
            copies.append(
                pltpu.async_copy(
                    table_hbm.at[idx_v.at[pl.ds(j * _CHUNK, _CHUNK)]],
                    rows_v.at[pl.ds(j * _CHUNK, _CHUNK)],
                    sem,
                )
            )
        for c in copies:
            c.wait()
        pltpu.sync_copy(rows_v, out_hbm.at[pl.ds(base, bpw)])

    return gather_kernel(table, idx)


_TB = 1024


def _mlp_body(d_ref, r_ref, n_ref, w_ref, w1d, w1r, w1n, w1w, b1r, w2, b2r,
              w3r, b3r, o_ref):
    x = (
        jnp.dot(d_ref[...], w1d[...], preferred_element_type=jnp.float32)
        + jnp.dot(r_ref[0], w1r[...], preferred_element_type=jnp.float32)
        + jnp.dot(n_ref[0], w1n[...], preferred_element_type=jnp.float32)
        + jnp.dot(w_ref[0], w1w[...], preferred_element_type=jnp.float32)
        + b1r[...]
    )
    h1 = jnp.maximum(x, 0.0)
    h2 = jnp.maximum(
        jnp.dot(h1, w2[...], preferred_element_type=jnp.float32) + b2r[...], 0.0
    )
    o_ref[...] = jnp.sum(h2 * w3r[...], axis=1) + b3r[0, 0]


def _mlp(dense_feats, emb3, w1d, w1r, w1n, w1w, b1r, w2, b2r, w3r, b3r):
    grid = _B // _TB
    full = lambda shape: pl.BlockSpec(shape, lambda i: (0,) * len(shape))
    emb_spec = lambda k: pl.BlockSpec((1, _TB, _D), lambda i, _k=k: (_k, i, 0))
    return pl.pallas_call(
        _mlp_body,
        grid=(grid,),
        in_specs=[
            pl.BlockSpec((_TB, 7), lambda i: (i, 0)),
            emb_spec(0),
            emb_spec(1),
            emb_spec(2),
            full((7, 128)),
            full((_D, 128)),
            full((_D, 128)),
            full((_D, 128)),
            full((1, 128)),
            full((128, 64)),
            full((1, 64)),
            full((1, 64)),
            full((1, 1)),
        ],
        out_specs=pl.BlockSpec((_TB,), lambda i: (i,)),
        out_shape=jax.ShapeDtypeStruct((_B,), jnp.float32),
    )(dense_feats, emb3, emb3, emb3, w1d, w1r, w1n, w1w, b1r, w2, b2r, w3r, b3r)


def kernel(route_id, node_id, weekday, dense_feats, route_table, node_table,
           weekday_table, W1, b1, W2, b2, W3, b3):
    route_id = route_id.astype(jnp.int32)
    node_id = node_id.astype(jnp.int32)
    weekday = weekday.astype(jnp.int32)

    n_route = route_table.shape[0]
    n_node = node_table.shape[0]

    table = jnp.concatenate(
        [
            jnp.pad(route_table, ((0, 0), (0, _D - route_table.shape[1]))),
            node_table,
            jnp.pad(weekday_table, ((0, 0), (0, _D - weekday_table.shape[1]))),
        ],
        axis=0,
    )
    idx = jnp.concatenate(
        [route_id, node_id + n_route, weekday + n_route + n_node]
    )

    emb = _sc_gather(table, idx)
    emb3 = emb.reshape(3, _B, _D)

    w1d = W1[0:7]
    w1r = jnp.zeros((_D, 128), jnp.float32).at[0:8].set(W1[7:15])
    w1n = W1[15:31]
    w1w = jnp.zeros((_D, 128), jnp.float32).at[0:2].set(W1[31:33])
    b1r = b1.reshape(1, 128)
    b2r = b2.reshape(1, 64)
    w3r = W3.reshape(1, 64)
    b3r = b3.reshape(1, 1)

    return _mlp(dense_feats, emb3, w1d, w1r, w1n, w1w, b1r, W2, b2r, w3r, b3r)


# speedup vs baseline: 2.3626x; 2.0147x over previous
import functools

import jax
import jax.numpy as jnp
from jax import lax
from jax.experimental import pallas as pl
from jax.experimental.pallas import tpu as pltpu
from jax.experimental.pallas import tpu_sc as plsc

_B = 16384
_NC = 2
_NS = 16
_NW = _NC * _NS
_CHUNK = 128
_IDX_ROWS = (3 * _B) // _CHUNK
_RPW = _IDX_ROWS // _NW
_D = 16


def _sc_gather(table, idx):
    mesh = plsc.VectorSubcoreMesh(core_axis_name="c", subcore_axis_name="s")
    bpw = _RPW * _CHUNK
    n_rows = table.shape[0]

    @functools.partial(
        pl.kernel,
        out_type=jax.ShapeDtypeStruct((_IDX_ROWS * _CHUNK, _D), jnp.float32),
        mesh=mesh,
        scratch_types=[
            pltpu.VMEM((bpw,), jnp.int32),
            pltpu.VMEM((bpw, _D), jnp.float32),
            pltpu.MemorySpace.VMEM_SHARED((n_rows, _D), jnp.float32),
            pltpu.SemaphoreType.DMA,
        ],
        compiler_params=pltpu.CompilerParams(use_tc_tiling_on_sc=False),
    )
    def gather_kernel(table_hbm, idx_hbm, out_hbm, idx_v, rows_v, shared_v,
                      sem):
        sid = lax.axis_index("s")
        wid = sid * _NC + lax.axis_index("c")
        base = wid * bpw
        idx_copy = pltpu.async_copy(idx_hbm.at[pl.ds(base, bpw)], idx_v, sem)

        @pl.when(sid == 0)
        def _stage():
            pltpu.sync_copy(table_hbm, shared_v)

        idx_copy.wait()
        plsc.subcore_barrier()
        copies = []
        for j in range(_RPW):
            copies.append(
                pltpu.async_copy(
                    shared_v.at[idx_v.at[pl.ds(j * _CHUNK, _CHUNK)]],
                    rows_v.at[pl.ds(j * _CHUNK, _CHUNK)],
                    sem,
                )
            )
        for c in copies:
            c.wait()
        pltpu.sync_copy(rows_v, out_hbm.at[pl.ds(base, bpw)])

    return gather_kernel(table, idx)


_TB = 1024


def _mlp_body(d_ref, r_ref, n_ref, w_ref, w1d, w1r, w1n, w1w, b1r, w2, b2r,
              w3r, b3r, o_ref):
    x = (
        jnp.dot(d_ref[...], w1d[...], preferred_element_type=jnp.float32)
        + jnp.dot(r_ref[0], w1r[...], preferred_element_type=jnp.float32)
        + jnp.dot(n_ref[0], w1n[...], preferred_element_type=jnp.float32)
        + jnp.dot(w_ref[0], w1w[...], preferred_element_type=jnp.float32)
        + b1r[...]
    )
    h1 = jnp.maximum(x, 0.0)
    h2 = jnp.maximum(
        jnp.dot(h1, w2[...], preferred_element_type=jnp.float32) + b2r[...], 0.0
    )
    o_ref[...] = jnp.sum(h2 * w3r[...], axis=1) + b3r[0, 0]


def _mlp(dense_feats, emb3, w1d, w1r, w1n, w1w, b1r, w2, b2r, w3r, b3r):
    grid = _B // _TB
    full = lambda shape: pl.BlockSpec(shape, lambda i: (0,) * len(shape))
    emb_spec = lambda k: pl.BlockSpec((1, _TB, _D), lambda i, _k=k: (_k, i, 0))
    return pl.pallas_call(
        _mlp_body,
        grid=(grid,),
        in_specs=[
            pl.BlockSpec((_TB, 7), lambda i: (i, 0)),
            emb_spec(0),
            emb_spec(1),
            emb_spec(2),
            full((7, 128)),
            full((_D, 128)),
            full((_D, 128)),
            full((_D, 128)),
            full((1, 128)),
            full((128, 64)),
            full((1, 64)),
            full((1, 64)),
            full((1, 1)),
        ],
        out_specs=pl.BlockSpec((_TB,), lambda i: (i,)),
        out_shape=jax.ShapeDtypeStruct((_B,), jnp.float32),
    )(dense_feats, emb3, emb3, emb3, w1d, w1r, w1n, w1w, b1r, w2, b2r, w3r, b3r)


def kernel(route_id, node_id, weekday, dense_feats, route_table, node_table,
           weekday_table, W1, b1, W2, b2, W3, b3):
    route_id = route_id.astype(jnp.int32)
    node_id = node_id.astype(jnp.int32)
    weekday = weekday.astype(jnp.int32)

    n_route = route_table.shape[0]
    n_node = node_table.shape[0]

    table = jnp.concatenate(
        [
            jnp.pad(route_table, ((0, 0), (0, _D - route_table.shape[1]))),
            node_table,
            jnp.pad(weekday_table, ((0, 0), (0, _D - weekday_table.shape[1]))),
        ],
        axis=0,
    )
    idx = jnp.concatenate(
        [route_id, node_id + n_route, weekday + n_route + n_node]
    )

    emb = _sc_gather(table, idx)
    emb3 = emb.reshape(3, _B, _D)

    w1d = W1[0:7]
    w1r = jnp.zeros((_D, 128), jnp.float32).at[0:8].set(W1[7:15])
    w1n = W1[15:31]
    w1w = jnp.zeros((_D, 128), jnp.float32).at[0:2].set(W1[31:33])
    b1r = b1.reshape(1, 128)
    b2r = b2.reshape(1, 64)
    w3r = W3.reshape(1, 64)
    b3r = b3.reshape(1, 1)

    return _mlp(dense_feats, emb3, w1d, w1r, w1n, w1w, b1r, W2, b2r, w3r, b3r)


# speedup vs baseline: 2.8827x; 1.2202x over previous
import functools

import jax
import jax.numpy as jnp
from jax import lax
from jax.experimental import pallas as pl
from jax.experimental.pallas import tpu as pltpu
from jax.experimental.pallas import tpu_sc as plsc

_B = 16384
_NC = 2
_NS = 16
_NW = _NC * _NS
_CHUNK = 128
_BPW = _B // _NW
_NCH = _BPW // _CHUNK


def _sc_gather(route_table, node_table, route_id, node_id):
    mesh = plsc.VectorSubcoreMesh(core_axis_name="c", subcore_axis_name="s")

    @functools.partial(
        pl.kernel,
        out_type=(
            jax.ShapeDtypeStruct((_B, 8), jnp.float32),
            jax.ShapeDtypeStruct((_B, 16), jnp.float32),
        ),
        mesh=mesh,
        scratch_types=[
            pltpu.VMEM((2 * _BPW,), jnp.int32),
            pltpu.VMEM((_BPW, 8), jnp.float32),
            pltpu.VMEM((_BPW, 16), jnp.float32),
            pltpu.MemorySpace.VMEM_SHARED(route_table.shape, jnp.float32),
            pltpu.MemorySpace.VMEM_SHARED(node_table.shape, jnp.float32),
            pltpu.SemaphoreType.DMA,
        ],
        compiler_params=pltpu.CompilerParams(use_tc_tiling_on_sc=False),
    )
    def gather_kernel(rtab_hbm, ntab_hbm, rid_hbm, nid_hbm, rout_hbm, nout_hbm,
                      idx_v, rrows_v, nrows_v, rtab_s, ntab_s, sem):
        sid = lax.axis_index("s")
        wid = sid * _NC + lax.axis_index("c")
        base = wid * _BPW
        ridx_copy = pltpu.async_copy(
            rid_hbm.at[pl.ds(base, _BPW)], idx_v.at[pl.ds(0, _BPW)], sem)
        nidx_copy = pltpu.async_copy(
            nid_hbm.at[pl.ds(base, _BPW)], idx_v.at[pl.ds(_BPW, _BPW)], sem)

        @pl.when(sid == 0)
        def _stage():
            pltpu.sync_copy(rtab_hbm, rtab_s)
            pltpu.sync_copy(ntab_hbm, ntab_s)

        ridx_copy.wait()
        nidx_copy.wait()
        plsc.subcore_barrier()
        copies = []
        for j in range(_NCH):
            copies.append(
                pltpu.async_copy(
                    rtab_s.at[idx_v.at[pl.ds(j * _CHUNK, _CHUNK)]],
                    rrows_v.at[pl.ds(j * _CHUNK, _CHUNK)],
                    sem,
                )
            )
            copies.append(
                pltpu.async_copy(
                    ntab_s.at[idx_v.at[pl.ds(_BPW + j * _CHUNK, _CHUNK)]],
                    nrows_v.at[pl.ds(j * _CHUNK, _CHUNK)],
                    sem,
                )
            )
        for c in copies:
            c.wait()
        pltpu.sync_copy(rrows_v, rout_hbm.at[pl.ds(base, _BPW)])
        pltpu.sync_copy(nrows_v, nout_hbm.at[pl.ds(base, _BPW)])

    return gather_kernel(route_table, node_table, route_id, node_id)


_TB = 2048


def _mlp_body(d_ref, r_ref, n_ref, wk_ref, wtab_ref, w1_ref, b1_ref, w2_ref,
              b2_ref, w3_ref, b3_ref, o_ref):
    w1wk = jnp.dot(wtab_ref[...], w1_ref[31:33],
                   preferred_element_type=jnp.float32)
    onehot = (wk_ref[...][:, None] ==
              lax.broadcasted_iota(jnp.int32, (_TB, 3), 1)).astype(jnp.float32)
    x = (
        jnp.dot(d_ref[...], w1_ref[0:7], preferred_element_type=jnp.float32)
        + jnp.dot(r_ref[...], w1_ref[7:15], preferred_element_type=jnp.float32)
        + jnp.dot(n_ref[...], w1_ref[15:31], preferred_element_type=jnp.float32)
        + jnp.dot(onehot, w1wk, preferred_element_type=jnp.float32)
        + b1_ref[...]
    )
    h1 = jnp.maximum(x, 0.0)
    h2 = jnp.maximum(
        jnp.dot(h1, w2_ref[...], preferred_element_type=jnp.float32)
        + b2_ref[...], 0.0,
    )
    out = jnp.dot(h2, w3_ref[...], preferred_element_type=jnp.float32)
    o_ref[...] = out[:, 0] + b3_ref[0]


def _mlp(dense_feats, r_emb, n_emb, weekday, weekday_table, W1, b1, W2, b2,
         W3, b3):
    grid = _B // _TB
    full = lambda shape: pl.BlockSpec(shape, lambda i: (0,) * len(shape))
    return pl.pallas_call(
        _mlp_body,
        grid=(grid,),
        in_specs=[
            pl.BlockSpec((_TB, 7), lambda i: (i, 0)),
            pl.BlockSpec((_TB, 8), lambda i: (i, 0)),
            pl.BlockSpec((_TB, 16), lambda i: (i, 0)),
            pl.BlockSpec((_TB,), lambda i: (i,)),
            full((3, 2)),
            full((33, 128)),
            full((128,)),
            full((128, 64)),
            full((64,)),
            full((64, 1)),
            full((1,)),
        ],
        out_specs=pl.BlockSpec((_TB,), lambda i: (i,)),
        out_shape=jax.ShapeDtypeStruct((_B,), jnp.float32),
    )(dense_feats, r_emb, n_emb, weekday, weekday_table, W1, b1, W2, b2, W3,
      b3)


def kernel(route_id, node_id, weekday, dense_feats, route_table, node_table,
           weekday_table, W1, b1, W2, b2, W3, b3):
    route_id = route_id.astype(jnp.int32)
    node_id = node_id.astype(jnp.int32)
    weekday = weekday.astype(jnp.int32)

    r_emb, n_emb = _sc_gather(route_table, node_table, route_id, node_id)
    return _mlp(dense_feats, r_emb, n_emb, weekday, weekday_table, W1, b1, W2,
                b2, W3, b3)


# speedup vs baseline: 3.7864x; 1.3135x over previous
import functools

import jax
import jax.numpy as jnp
from jax import lax
from jax.experimental import pallas as pl
from jax.experimental.pallas import tpu as pltpu
from jax.experimental.pallas import tpu_sc as plsc

_B = 16384
_NC = 2
_NS = 16
_NW = _NC * _NS
_CHUNK = 128
_BPW = _B // _NW
_NCH = _BPW // _CHUNK


def _sc_gather(route_table, node_table, route_id, node_id):
    mesh = plsc.VectorSubcoreMesh(core_axis_name="c", subcore_axis_name="s")

    @functools.partial(
        pl.kernel,
        out_type=jax.ShapeDtypeStruct((_B, 128), jnp.float32),
        mesh=mesh,
        scratch_types=[
            pltpu.VMEM((2 * _BPW,), jnp.int32),
            pltpu.VMEM((_BPW, 8), jnp.float32),
            pltpu.VMEM((_BPW, 16), jnp.float32),
            pltpu.MemorySpace.VMEM_SHARED(route_table.shape, jnp.float32),
            pltpu.MemorySpace.VMEM_SHARED(node_table.shape, jnp.float32),
            pltpu.SemaphoreType.DMA,
        ],
        compiler_params=pltpu.CompilerParams(use_tc_tiling_on_sc=False),
    )
    def gather_kernel(rtab_hbm, ntab_hbm, rid_hbm, nid_hbm, out_hbm,
                      idx_v, rrows_v, nrows_v, rtab_s, ntab_s, sem):
        sid = lax.axis_index("s")
        wid = sid * _NC + lax.axis_index("c")
        base = wid * _BPW
        ridx_copy = pltpu.async_copy(
            rid_hbm.at[pl.ds(base, _BPW)], idx_v.at[pl.ds(0, _BPW)], sem)
        nidx_copy = pltpu.async_copy(
            nid_hbm.at[pl.ds(base, _BPW)], idx_v.at[pl.ds(_BPW, _BPW)], sem)

        @pl.when(sid == 0)
        def _stage():
            pltpu.sync_copy(rtab_hbm, rtab_s)
            pltpu.sync_copy(ntab_hbm, ntab_s)

        ridx_copy.wait()
        nidx_copy.wait()
        plsc.subcore_barrier()
        copies = []
        for j in range(_NCH):
            copies.append(
                pltpu.async_copy(
                    rtab_s.at[idx_v.at[pl.ds(j * _CHUNK, _CHUNK)]],
                    rrows_v.at[pl.ds(j * _CHUNK, _CHUNK)],
                    sem,
                )
            )
            copies.append(
                pltpu.async_copy(
                    ntab_s.at[idx_v.at[pl.ds(_BPW + j * _CHUNK, _CHUNK)]],
                    nrows_v.at[pl.ds(j * _CHUNK, _CHUNK)],
                    sem,
                )
            )
        for c in copies:
            c.wait()
        pltpu.sync_copy(rrows_v,
                        out_hbm.at[pl.ds(base, _BPW), pl.ds(8, 8)])
        pltpu.sync_copy(nrows_v,
                        out_hbm.at[pl.ds(base, _BPW), pl.ds(16, 16)])

    return gather_kernel(route_table, node_table, route_id, node_id)


_TB = 4096


def _mlp_body(d_ref, p_ref, wk_ref, wtab_ref, w1_ref, b1_ref, w2_ref,
              b2_ref, w3_ref, b3_ref, o_ref):
    w1wk = jnp.dot(wtab_ref[...], w1_ref[31:33],
                   preferred_element_type=jnp.float32)
    onehot = (wk_ref[...][:, None] ==
              lax.broadcasted_iota(jnp.int32, (_TB, 3), 1)).astype(jnp.float32)
    x = (
        jnp.dot(d_ref[...], w1_ref[0:7], preferred_element_type=jnp.float32)
        + jnp.dot(p_ref[...][:, 8:32], w1_ref[7:31],
                  preferred_element_type=jnp.float32)
        + jnp.dot(onehot, w1wk, preferred_element_type=jnp.float32)
        + b1_ref[...]
    )
    h1 = jnp.maximum(x, 0.0)
    h2 = jnp.maximum(
        jnp.dot(h1, w2_ref[...], preferred_element_type=jnp.float32)
        + b2_ref[...], 0.0,
    )
    out = jnp.dot(h2, w3_ref[...], preferred_element_type=jnp.float32)
    o_ref[...] = out[:, 0] + b3_ref[0]


def _mlp(dense_feats, pack, weekday, weekday_table, W1, b1, W2, b2, W3, b3):
    grid = _B // _TB
    full = lambda shape: pl.BlockSpec(shape, lambda i: (0,) * len(shape))
    return pl.pallas_call(
        _mlp_body,
        grid=(grid,),
        in_specs=[
            pl.BlockSpec((_TB, 7), lambda i: (i, 0)),
            pl.BlockSpec((_TB, 128), lambda i: (i, 0)),
            pl.BlockSpec((_TB,), lambda i: (i,)),
            full((3, 2)),
            full((33, 128)),
            full((128,)),
            full((128, 64)),
            full((64,)),
            full((64, 1)),
            full((1,)),
        ],
        out_specs=pl.BlockSpec((_TB,), lambda i: (i,)),
        out_shape=jax.ShapeDtypeStruct((_B,), jnp.float32),
    )(dense_feats, pack, weekday, weekday_table, W1, b1, W2, b2, W3, b3)


def kernel(route_id, node_id, weekday, dense_feats, route_table, node_table,
           weekday_table, W1, b1, W2, b2, W3, b3):
    route_id = route_id.astype(jnp.int32)
    node_id = node_id.astype(jnp.int32)
    weekday = weekday.astype(jnp.int32)

    pack = _sc_gather(route_table, node_table, route_id, node_id)
    return _mlp(dense_feats, pack, weekday, weekday_table, W1, b1, W2, b2,
                W3, b3)
